# baseline (speedup 1.0000x reference)
import jax
import jax.numpy as jnp
from jax.experimental import pallas as pl


def _dpmean_kernel(f_ref, mu_ref, dist_ref, idx_ref, maxd_ref):
    i = pl.program_id(0)
    f = f_ref[...]
    m = mu_ref[...]
    dot = jax.lax.dot_general(
        f, m, (((1,), (1,)), ((), ())),
        preferred_element_type=jnp.float32)
    f2 = jnp.sum(f * f, axis=1, keepdims=True)
    mu2 = jnp.sum(m * m, axis=1)
    d2 = f2 + mu2[None, :] - 2.0 * dot
    mind2 = jnp.min(d2, axis=1, keepdims=True)
    k = d2.shape[1]
    iota = jax.lax.broadcasted_iota(jnp.int32, d2.shape, 1)
    idx = jnp.min(jnp.where(d2 == mind2, iota, k), axis=1, keepdims=True)
    dist = jnp.sqrt(jnp.maximum(mind2, 0.0))
    dist_ref[...] = dist
    idx_ref[...] = idx
    local = jnp.max(dist, axis=0, keepdims=True)

    @pl.when(i == 0)
    def _init():
        maxd_ref[...] = local

    @pl.when(i > 0)
    def _acc():
        maxd_ref[...] = jnp.maximum(maxd_ref[...], local)


def kernel(features, mu):
    f = features[:, 0, :]
    b, d = f.shape
    kk = mu.shape[0]
    bb = 256
    dist, idx, maxd = pl.pallas_call(
        _dpmean_kernel,
        grid=(b // bb,),
        in_specs=[
            pl.BlockSpec((bb, d), lambda i: (i, 0)),
            pl.BlockSpec((kk, d), lambda i: (0, 0)),
        ],
        out_specs=[
            pl.BlockSpec((bb, 1), lambda i: (i, 0)),
            pl.BlockSpec((bb, 1), lambda i: (i, 0)),
            pl.BlockSpec((1, 1), lambda i: (0, 0)),
        ],
        out_shape=[
            jax.ShapeDtypeStruct((b, 1), jnp.float32),
            jax.ShapeDtypeStruct((b, 1), jnp.int32),
            jax.ShapeDtypeStruct((1, 1), jnp.float32),
        ],
    )(f, mu)
    return dist[:, 0], idx[:, 0], maxd[0]


# speedup vs baseline: 1.2077x; 1.2077x over previous
import jax
import jax.numpy as jnp
from jax.experimental import pallas as pl


def _dpmean_kernel(f_ref, mu_ref, dist_ref, idx_ref, maxd_ref):
    f = f_ref[...]
    m = mu_ref[...]
    ndot = jax.lax.dot_general(
        -2.0 * f, m, (((1,), (1,)), ((), ())),
        preferred_element_type=jnp.float32)
    f2 = jnp.sum(f * f, axis=1, keepdims=True)
    mu2 = jnp.sum(m * m, axis=1)
    t = f2 + mu2[None, :]
    d2 = t + ndot
    mind2 = jnp.min(d2, axis=1, keepdims=True)
    k = d2.shape[1]
    iota = jax.lax.broadcasted_iota(jnp.int32, d2.shape, 1)
    idx = jnp.min(jnp.where(d2 == mind2, iota, k), axis=1, keepdims=True)
    dist = jnp.sqrt(jnp.maximum(mind2, 0.0))
    dist_ref[...] = dist
    idx_ref[...] = idx
    maxd_ref[...] = jnp.max(dist, axis=0, keepdims=True)


def kernel(features, mu):
    f = features[:, 0, :]
    b = f.shape[0]
    dist, idx, maxd = pl.pallas_call(
        _dpmean_kernel,
        out_shape=[
            jax.ShapeDtypeStruct((b, 1), jnp.float32),
            jax.ShapeDtypeStruct((b, 1), jnp.int32),
            jax.ShapeDtypeStruct((1, 1), jnp.float32),
        ],
    )(f, mu)
    return dist[:, 0], idx[:, 0], maxd[0]


# speedup vs baseline: 1.6258x; 1.3462x over previous
import jax
import jax.numpy as jnp
from jax.experimental import pallas as pl


def _dpmean_kernel(f_ref, mu_ref, dist_ref, idx_ref, maxd_ref):
    i = pl.program_id(0)
    f = f_ref[...]
    m = mu_ref[...]
    ndot = jax.lax.dot_general(
        -2.0 * f, m, (((1,), (1,)), ((), ())),
        preferred_element_type=jnp.float32)
    f2 = jnp.sum(f * f, axis=1, keepdims=True)
    mu2 = jnp.sum(m * m, axis=1)
    t = f2 + mu2[None, :]
    d2 = t + ndot
    mind2 = jnp.min(d2, axis=1, keepdims=True)
    k = d2.shape[1]
    iota = jax.lax.broadcasted_iota(jnp.int32, (1, k), 1)
    idx = jnp.min(jnp.where(d2 == mind2, iota, k), axis=1, keepdims=True)
    dist = jnp.sqrt(jnp.maximum(mind2, 0.0))
    dist_ref[...] = jnp.reshape(dist, dist_ref.shape)
    idx_ref[...] = jnp.reshape(idx, idx_ref.shape)
    local = jnp.max(dist, axis=0, keepdims=True)

    @pl.when(i == 0)
    def _init():
        maxd_ref[...] = local

    @pl.when(i > 0)
    def _acc():
        maxd_ref[...] = jnp.maximum(maxd_ref[...], local)


def kernel(features, mu):
    f = features[:, 0, :]
    b, d = f.shape
    kk = mu.shape[0]
    nsteps = 2
    bb = b // nsteps
    rb = bb // 128
    dist, idx, maxd = pl.pallas_call(
        _dpmean_kernel,
        grid=(nsteps,),
        in_specs=[
            pl.BlockSpec((bb, d), lambda i: (i, 0)),
            pl.BlockSpec((kk, d), lambda i: (0, 0)),
        ],
        out_specs=[
            pl.BlockSpec((rb, 128), lambda i: (i, 0)),
            pl.BlockSpec((rb, 128), lambda i: (i, 0)),
            pl.BlockSpec((1, 1), lambda i: (0, 0)),
        ],
        out_shape=[
            jax.ShapeDtypeStruct((b // 128, 128), jnp.float32),
            jax.ShapeDtypeStruct((b // 128, 128), jnp.int32),
            jax.ShapeDtypeStruct((1, 1), jnp.float32),
        ],
    )(f, mu)
    return dist.reshape(b), idx.reshape(b), maxd.reshape(1)
